# baseline (speedup 1.0000x reference)
import functools

import jax
import jax.numpy as jnp
from jax import lax
from jax.experimental import pallas as pl
from jax.experimental.pallas import tpu as pltpu
from jax.experimental.pallas import tpu_sc as plsc

B, L = 1024, 200
B_TOT = B * L
HID = 96
D = 32
NC, NS = 2, 16
NW = NC * NS
BPW = B_TOT // NW
BM = 1024

N_TESTS, N_QUESTIONS, N_TAGS = 10000, 100000, 1000
V_TEST, V_Q, V_TAG = N_TESTS + 1, N_QUESTIONS + 1, N_TAGS + 1
OFF_Q = V_TEST
OFF_TAG = V_TEST + V_Q
V_ALL = V_TEST + V_Q + V_TAG
V_PAD = ((V_ALL + 8 * NS - 1) // (8 * NS)) * (8 * NS)
RPS = V_PAD // NS


def _sc_gather(tab_all, idx):
    mesh = plsc.VectorSubcoreMesh(core_axis_name="c", subcore_axis_name="s")
    C = 320
    NCHUNK = BPW // C

    @functools.partial(
        pl.kernel,
        mesh=mesh,
        compiler_params=pltpu.CompilerParams(use_tc_tiling_on_sc=False),
        out_type=jax.ShapeDtypeStruct((3, B_TOT, D), jnp.bfloat16),
        scratch_types=[
            pltpu.VMEM_SHARED((V_PAD, D), jnp.bfloat16),
            pltpu.VMEM((3, C), jnp.int32),
            pltpu.VMEM((3, C, D), jnp.bfloat16),
            pltpu.SemaphoreType.DMA,
        ],
    )
    def k(tab_hbm, idx_hbm, out, tab_s, idx_v, rows_v, sem):
        sid = lax.axis_index("s")
        wid = sid * NC + lax.axis_index("c")
        base = wid * BPW
        pltpu.sync_copy(tab_hbm.at[pl.ds(sid * RPS, RPS)],
                        tab_s.at[pl.ds(sid * RPS, RPS)])
        plsc.subcore_barrier()

        def chunk(c, carry):
            cb = base + c * C
            for t in range(3):
                pltpu.sync_copy(idx_hbm.at[t, pl.ds(cb, C)], idx_v.at[t])
            handles = [
                pltpu.async_copy(tab_s.at[idx_v.at[t]], rows_v.at[t], sem)
                for t in range(3)
            ]
            for t in range(3):
                handles[t].wait()
                pltpu.sync_copy(rows_v.at[t], out.at[t, pl.ds(cb, C)])
            return carry

        lax.fori_loop(0, NCHUNK, chunk, 0)

    return k(tab_all, idx)


def _tc_proj(g, ii3, tab_int, w4, b_proj, ln_w, ln_b):
    def body(g_ref, ii_ref, ti_ref, w_ref, b_ref, lnw_ref, lnb_ref, o_ref):
        m = lax.dot_general(ti_ref[...], w_ref[0], (((1,), (0,)), ((), ())),
                            preferred_element_type=jnp.float32)
        m_bf = m.astype(jnp.bfloat16)
        R = BM // 8
        cols = lax.broadcasted_iota(jnp.int32, (R, 8), 1)
        for s in range(8):
            ii = ii_ref[0][:, s:s + 1]
            oh = (ii == cols).astype(jnp.bfloat16)
            acc = lax.dot_general(
                oh, m_bf, (((1,), (0,)), ((), ())),
                preferred_element_type=jnp.float32)
            for t in range(3):
                acc = acc + lax.dot_general(
                    g_ref[t][:, s * D:(s + 1) * D], w_ref[t + 1],
                    (((1,), (0,)), ((), ())),
                    preferred_element_type=jnp.float32)
            x = acc + b_ref[0]
            mu = jnp.mean(x, axis=1, keepdims=True)
            xc = x - mu
            var = jnp.mean(xc * xc, axis=1, keepdims=True)
            o_ref[pl.Slice(s, R, 8), :] = (
                xc * lax.rsqrt(var + 1e-5) * lnw_ref[0] + lnb_ref[0])

    return pl.pallas_call(
        body,
        grid=(B_TOT // BM,),
        in_specs=[
            pl.BlockSpec((3, BM // 8, 8 * D), lambda i: (0, i, 0)),
            pl.BlockSpec((1, BM // 8, 8), lambda i: (i, 0, 0)),
            pl.BlockSpec((8, D), lambda i: (0, 0)),
            pl.BlockSpec((4, D, HID), lambda i: (0, 0, 0)),
            pl.BlockSpec((1, HID), lambda i: (0, 0)),
            pl.BlockSpec((1, HID), lambda i: (0, 0)),
            pl.BlockSpec((1, HID), lambda i: (0, 0)),
        ],
        out_specs=pl.BlockSpec((BM, HID), lambda i: (i, 0)),
        out_shape=jax.ShapeDtypeStruct((B_TOT, HID), jnp.float32),
    )(g, ii3, tab_int, w4, b_proj.reshape(1, HID), ln_w.reshape(1, HID),
      ln_b.reshape(1, HID))


def kernel(interaction, test, question, tag, correct, mask,
           emb_interaction, emb_test, emb_question, emb_tag,
           W_proj, b_proj, ln_w, ln_b):
    tab_all = jnp.concatenate([
        emb_test.astype(jnp.bfloat16),
        emb_question.astype(jnp.bfloat16),
        emb_tag.astype(jnp.bfloat16),
        jnp.zeros((V_PAD - V_ALL, D), jnp.bfloat16),
    ])
    idx = jnp.stack([
        test.reshape(-1),
        question.reshape(-1) + OFF_Q,
        tag.reshape(-1) + OFF_TAG,
    ]).astype(jnp.int32)
    g = _sc_gather(tab_all, idx).reshape(3, B_TOT // 8, 8 * D)
    ii3 = interaction.astype(jnp.int32).reshape(B_TOT // BM, BM // 8, 8)
    tab_int = jnp.zeros((8, D), jnp.bfloat16).at[:3].set(
        emb_interaction.astype(jnp.bfloat16))
    w4 = W_proj.reshape(HID, 4, D).transpose(1, 2, 0).astype(jnp.bfloat16)
    x = _tc_proj(g, ii3, tab_int, w4, b_proj, ln_w, ln_b)
    return (x.reshape(B, L, HID), interaction.shape[0])


# speedup vs baseline: 1.0853x; 1.0853x over previous
import functools

import jax
import jax.numpy as jnp
from jax import lax
from jax.experimental import pallas as pl
from jax.experimental.pallas import tpu as pltpu
from jax.experimental.pallas import tpu_sc as plsc

B, L = 1024, 200
B_TOT = B * L
HID = 96
D = 32
NC, NS = 2, 16
NW = NC * NS
BPW = B_TOT // NW
BM = 2048

N_TESTS, N_QUESTIONS, N_TAGS = 10000, 100000, 1000
V_TEST, V_Q, V_TAG = N_TESTS + 1, N_QUESTIONS + 1, N_TAGS + 1
OFF_Q = V_TEST
OFF_TAG = V_TEST + V_Q
V_ALL = V_TEST + V_Q + V_TAG
V_PAD = ((V_ALL + 8 * NS - 1) // (8 * NS)) * (8 * NS)
RPS = V_PAD // NS


def _sc_gather(tab_all, idx):
    mesh = plsc.VectorSubcoreMesh(core_axis_name="c", subcore_axis_name="s")
    C = 320
    NCHUNK = BPW // C

    @functools.partial(
        pl.kernel,
        mesh=mesh,
        compiler_params=pltpu.CompilerParams(use_tc_tiling_on_sc=False),
        out_type=jax.ShapeDtypeStruct((3, B_TOT, D), jnp.bfloat16),
        scratch_types=[
            pltpu.VMEM_SHARED((V_PAD, D), jnp.bfloat16),
            pltpu.VMEM((3, C), jnp.int32),
            pltpu.VMEM((3, C, D), jnp.bfloat16),
            pltpu.SemaphoreType.DMA,
        ],
    )
    def k(tab_hbm, idx_hbm, out, tab_s, idx_v, rows_v, sem):
        sid = lax.axis_index("s")
        wid = sid * NC + lax.axis_index("c")
        base = wid * BPW
        pltpu.sync_copy(tab_hbm.at[pl.ds(sid * RPS, RPS)],
                        tab_s.at[pl.ds(sid * RPS, RPS)])
        plsc.subcore_barrier()

        def chunk(c, carry):
            cb = base + c * C
            for t in range(3):
                pltpu.sync_copy(idx_hbm.at[t, pl.ds(cb, C)], idx_v.at[t])
            handles = [
                pltpu.async_copy(tab_s.at[idx_v.at[t]], rows_v.at[t], sem)
                for t in range(3)
            ]
            for t in range(3):
                handles[t].wait()
                pltpu.sync_copy(rows_v.at[t], out.at[t, pl.ds(cb, C)])
            return carry

        lax.fori_loop(0, NCHUNK, chunk, 0)

    return k(tab_all, idx)


def _tc_proj(g, ii3, tab_int, w4, b_proj, ln_w, ln_b):
    def body(g_ref, ii_ref, ti_ref, w_ref, b_ref, lnw_ref, lnb_ref, o_ref):
        m = lax.dot_general(ti_ref[...], w_ref[0], (((1,), (0,)), ((), ())),
                            preferred_element_type=jnp.float32)
        m_bf = m.astype(jnp.bfloat16)
        R = BM // 8
        cols = lax.broadcasted_iota(jnp.int32, (R, 8), 1)
        for s in range(8):
            ii = ii_ref[0, s][:, None]
            oh = (ii == cols).astype(jnp.bfloat16)
            acc = lax.dot_general(
                oh, m_bf, (((1,), (0,)), ((), ())),
                preferred_element_type=jnp.float32)
            for t in range(3):
                acc = acc + lax.dot_general(
                    g_ref[t][:, s * D:(s + 1) * D], w_ref[t + 1],
                    (((1,), (0,)), ((), ())),
                    preferred_element_type=jnp.float32)
            x = acc + b_ref[0]
            mu = jnp.mean(x, axis=1, keepdims=True)
            xc = x - mu
            var = jnp.mean(xc * xc, axis=1, keepdims=True)
            o_ref[pl.Slice(s, R, 8), :] = (
                xc * lax.rsqrt(var + 1e-5) * lnw_ref[0] + lnb_ref[0])

    return pl.pallas_call(
        body,
        grid=(B_TOT // BM,),
        in_specs=[
            pl.BlockSpec((3, BM // 8, 8 * D), lambda i: (0, i, 0)),
            pl.BlockSpec((1, 8, BM // 8), lambda i: (i, 0, 0)),
            pl.BlockSpec((8, D), lambda i: (0, 0)),
            pl.BlockSpec((4, D, HID), lambda i: (0, 0, 0)),
            pl.BlockSpec((1, HID), lambda i: (0, 0)),
            pl.BlockSpec((1, HID), lambda i: (0, 0)),
            pl.BlockSpec((1, HID), lambda i: (0, 0)),
        ],
        out_specs=pl.BlockSpec((BM, HID), lambda i: (i, 0)),
        out_shape=jax.ShapeDtypeStruct((B_TOT, HID), jnp.float32),
    )(g, ii3, tab_int, w4, b_proj.reshape(1, HID), ln_w.reshape(1, HID),
      ln_b.reshape(1, HID))


def kernel(interaction, test, question, tag, correct, mask,
           emb_interaction, emb_test, emb_question, emb_tag,
           W_proj, b_proj, ln_w, ln_b):
    tab_all = jnp.concatenate([
        emb_test.astype(jnp.bfloat16),
        emb_question.astype(jnp.bfloat16),
        emb_tag.astype(jnp.bfloat16),
        jnp.zeros((V_PAD - V_ALL, D), jnp.bfloat16),
    ])
    idx = jnp.stack([
        test.reshape(-1),
        question.reshape(-1) + OFF_Q,
        tag.reshape(-1) + OFF_TAG,
    ]).astype(jnp.int32)
    g = _sc_gather(tab_all, idx).reshape(3, B_TOT // 8, 8 * D)
    ii3 = interaction.astype(jnp.int32).reshape(
        B_TOT // BM, BM // 8, 8).transpose(0, 2, 1)
    tab_int = jnp.zeros((8, D), jnp.bfloat16).at[:3].set(
        emb_interaction.astype(jnp.bfloat16))
    w4 = W_proj.reshape(HID, 4, D).transpose(1, 2, 0).astype(jnp.bfloat16)
    x = _tc_proj(g, ii3, tab_int, w4, b_proj, ln_w, ln_b)
    return (x.reshape(B, L, HID), interaction.shape[0])


# speedup vs baseline: 1.1043x; 1.0175x over previous
import functools

import jax
import jax.numpy as jnp
from jax import lax
from jax.experimental import pallas as pl
from jax.experimental.pallas import tpu as pltpu
from jax.experimental.pallas import tpu_sc as plsc

B, L = 1024, 200
B_TOT = B * L
HID = 96
D = 32
NC, NS = 2, 16
NW = NC * NS
BPW = B_TOT // NW
BM = 4096

N_TESTS, N_QUESTIONS, N_TAGS = 10000, 100000, 1000
V_TEST, V_Q, V_TAG = N_TESTS + 1, N_QUESTIONS + 1, N_TAGS + 1
OFF_Q = V_TEST
OFF_TAG = V_TEST + V_Q
V_ALL = V_TEST + V_Q + V_TAG
V_PAD = ((V_ALL + 8 * NS - 1) // (8 * NS)) * (8 * NS)
RPS = V_PAD // NS


def _sc_gather(tab_all, idx):
    mesh = plsc.VectorSubcoreMesh(core_axis_name="c", subcore_axis_name="s")
    C = 320
    NCHUNK = BPW // C

    @functools.partial(
        pl.kernel,
        mesh=mesh,
        compiler_params=pltpu.CompilerParams(use_tc_tiling_on_sc=False),
        out_type=jax.ShapeDtypeStruct((3, B_TOT, D), jnp.bfloat16),
        scratch_types=[
            pltpu.VMEM_SHARED((V_PAD, D), jnp.bfloat16),
            pltpu.VMEM((3, C), jnp.int32),
            pltpu.VMEM((3, C, D), jnp.bfloat16),
            pltpu.SemaphoreType.DMA,
        ],
    )
    def k(tab_hbm, idx_hbm, out, tab_s, idx_v, rows_v, sem):
        sid = lax.axis_index("s")
        wid = sid * NC + lax.axis_index("c")
        base = wid * BPW
        pltpu.sync_copy(tab_hbm.at[pl.ds(sid * RPS, RPS)],
                        tab_s.at[pl.ds(sid * RPS, RPS)])
        plsc.subcore_barrier()

        def chunk(c, carry):
            cb = base + c * C
            for t in range(3):
                pltpu.sync_copy(idx_hbm.at[t, pl.ds(cb, C)], idx_v.at[t])
            handles = [
                pltpu.async_copy(tab_s.at[idx_v.at[t]], rows_v.at[t], sem)
                for t in range(3)
            ]
            for t in range(3):
                handles[t].wait()
                pltpu.sync_copy(rows_v.at[t], out.at[t, pl.ds(cb, C)])
            return carry

        lax.fori_loop(0, NCHUNK, chunk, 0)

    return k(tab_all, idx)


def _tc_proj(g, ii3, tab_int, w4, b_proj, ln_w, ln_b):
    def body(g_ref, ii_ref, ti_ref, w_ref, b_ref, lnw_ref, lnb_ref, o_ref):
        m = lax.dot_general(ti_ref[...], w_ref[0], (((1,), (0,)), ((), ())),
                            preferred_element_type=jnp.float32)
        m_bf = m.astype(jnp.bfloat16)
        R = BM // 8
        cols = lax.broadcasted_iota(jnp.int32, (R, 8), 1)
        for s in range(8):
            ii = ii_ref[0, s][:, None]
            oh = (ii == cols).astype(jnp.bfloat16)
            acc = lax.dot_general(
                oh, m_bf, (((1,), (0,)), ((), ())),
                preferred_element_type=jnp.float32)
            for t in range(3):
                acc = acc + lax.dot_general(
                    g_ref[t][:, s * D:(s + 1) * D], w_ref[t + 1],
                    (((1,), (0,)), ((), ())),
                    preferred_element_type=jnp.float32)
            x = acc + b_ref[0]
            mu = jnp.mean(x, axis=1, keepdims=True)
            xc = x - mu
            var = jnp.mean(xc * xc, axis=1, keepdims=True)
            o_ref[pl.Slice(s, R, 8), :] = (
                xc * lax.rsqrt(var + 1e-5) * lnw_ref[0] + lnb_ref[0])

    return pl.pallas_call(
        body,
        grid=(B_TOT // BM,),
        in_specs=[
            pl.BlockSpec((3, BM // 8, 8 * D), lambda i: (0, i, 0)),
            pl.BlockSpec((1, 8, BM // 8), lambda i: (i, 0, 0)),
            pl.BlockSpec((8, D), lambda i: (0, 0)),
            pl.BlockSpec((4, D, HID), lambda i: (0, 0, 0)),
            pl.BlockSpec((1, HID), lambda i: (0, 0)),
            pl.BlockSpec((1, HID), lambda i: (0, 0)),
            pl.BlockSpec((1, HID), lambda i: (0, 0)),
        ],
        out_specs=pl.BlockSpec((BM, HID), lambda i: (i, 0)),
        out_shape=jax.ShapeDtypeStruct((B_TOT, HID), jnp.float32),
    )(g, ii3, tab_int, w4, b_proj.reshape(1, HID), ln_w.reshape(1, HID),
      ln_b.reshape(1, HID))


def kernel(interaction, test, question, tag, correct, mask,
           emb_interaction, emb_test, emb_question, emb_tag,
           W_proj, b_proj, ln_w, ln_b):
    tab_all = jnp.concatenate([
        emb_test.astype(jnp.bfloat16),
        emb_question.astype(jnp.bfloat16),
        emb_tag.astype(jnp.bfloat16),
        jnp.zeros((V_PAD - V_ALL, D), jnp.bfloat16),
    ])
    idx = jnp.stack([
        test.reshape(-1),
        question.reshape(-1) + OFF_Q,
        tag.reshape(-1) + OFF_TAG,
    ]).astype(jnp.int32)
    g = _sc_gather(tab_all, idx).reshape(3, B_TOT // 8, 8 * D)
    ii3 = interaction.astype(jnp.int32).reshape(
        B_TOT // BM, BM // 8, 8).transpose(0, 2, 1)
    tab_int = jnp.zeros((8, D), jnp.bfloat16).at[:3].set(
        emb_interaction.astype(jnp.bfloat16))
    w4 = W_proj.reshape(HID, 4, D).transpose(1, 2, 0).astype(jnp.bfloat16)
    x = _tc_proj(g, ii3, tab_int, w4, b_proj, ln_w, ln_b)
    return (x.reshape(B, L, HID), interaction.shape[0])
